# Initial kernel scaffold; baseline (speedup 1.0000x reference)
import jax
import jax.numpy as jnp
from jax.experimental import pallas as pl


def kernel(x, edge_index, Wa, Wb, Wc, L):
    raise NotImplementedError("write your pallas kernel here")


# speedup vs baseline: 5.4908x; 5.4908x over previous
import functools

import jax
import jax.numpy as jnp
from jax import lax
from jax.experimental import pallas as pl
from jax.experimental.pallas import tpu as pltpu
from jax.experimental.pallas import tpu_sc as plsc

N_NODES = 10000
DIM = 128
N_EDGES = 320000

NC = 2
NS = 16
NW = NC * NS

EB = 128
NBLK = N_EDGES // EB
BLK_PER_TILE = -(-NBLK // NW)
ROWS_PER_TILE = 624
ROWS_TAIL = N_NODES - NS * ROWS_PER_TILE


def _segsum_body(x_hbm, src_hbm, dst_hbm, zeros_hbm, out_hbm,
                 idx_s, idx_d, rows, agg_sh, sem):
    c = lax.axis_index("c")
    s = lax.axis_index("s")
    wid = c * NS + s

    base = pl.multiple_of(s * ROWS_PER_TILE, 8)
    pltpu.sync_copy(zeros_hbm.at[pl.ds(base, ROWS_PER_TILE)],
                    agg_sh.at[pl.ds(base, ROWS_PER_TILE)])

    @pl.when(s == NS - 1)
    def _():
        pltpu.sync_copy(zeros_hbm.at[pl.ds(NS * ROWS_PER_TILE, ROWS_TAIL)],
                        agg_sh.at[pl.ds(NS * ROWS_PER_TILE, ROWS_TAIL)])

    plsc.subcore_barrier()

    def body(b, carry):
        blk = b * NW + wid

        @pl.when(blk < NBLK)
        def _():
            off = pl.multiple_of(blk * EB, EB)
            pltpu.sync_copy(src_hbm.at[pl.ds(off, EB)], idx_s)
            pltpu.sync_copy(dst_hbm.at[pl.ds(off, EB)], idx_d)
            pltpu.async_copy(x_hbm.at[idx_s], rows, sem).wait()
            pltpu.sync_copy(rows, agg_sh.at[idx_d], add=True)

        return carry

    lax.fori_loop(0, BLK_PER_TILE, body, 0)

    plsc.subcore_barrier()
    pltpu.sync_copy(agg_sh.at[pl.ds(base, ROWS_PER_TILE)],
                    out_hbm.at[c, pl.ds(base, ROWS_PER_TILE)])

    @pl.when(s == NS - 1)
    def _():
        pltpu.sync_copy(agg_sh.at[pl.ds(NS * ROWS_PER_TILE, ROWS_TAIL)],
                        out_hbm.at[c, pl.ds(NS * ROWS_PER_TILE, ROWS_TAIL)])


@jax.jit
def _sc_segsum(x, src, dst, zeros):
    mesh = plsc.VectorSubcoreMesh(core_axis_name="c", subcore_axis_name="s")
    return pl.kernel(
        _segsum_body,
        out_type=jax.ShapeDtypeStruct((NC, N_NODES, DIM), jnp.float32),
        mesh=mesh,
        scratch_types=[
            pltpu.VMEM((EB,), jnp.int32),
            pltpu.VMEM((EB,), jnp.int32),
            pltpu.VMEM((EB, DIM), jnp.float32),
            pltpu.VMEM_SHARED((N_NODES, DIM), jnp.float32),
            pltpu.SemaphoreType.DMA,
        ],
    )(x, src, dst, zeros)


ROW_BLK = 1000


def _mlp_body(h_ref, agg_ref, wa_ref, wb_ref, wc_ref, l_ref,
              hout_ref, head_ref, acc_ref):
    i = pl.program_id(0)
    z = h_ref[...] + agg_ref[0] + agg_ref[1]
    z = jnp.maximum(jnp.dot(z, wa_ref[...], preferred_element_type=jnp.float32), 0.0)
    z = jnp.maximum(jnp.dot(z, wb_ref[...], preferred_element_type=jnp.float32), 0.0)
    z = jnp.maximum(jnp.dot(z, wc_ref[...], preferred_element_type=jnp.float32), 0.0)
    hout_ref[...] = z

    @pl.when(i == 0)
    def _():
        acc_ref[...] = jnp.zeros_like(acc_ref)

    acc_ref[...] += jnp.sum(z, axis=0, keepdims=True)

    @pl.when(i == pl.num_programs(0) - 1)
    def _():
        head_ref[...] = jnp.dot(acc_ref[...] / N_NODES, l_ref[...],
                                preferred_element_type=jnp.float32)


def _final_body(h_ref, agg_ref, wa_ref, wb_ref, wc_ref, l_ref, prev_ref,
                out_ref, acc_ref):
    i = pl.program_id(0)
    z = h_ref[...] + agg_ref[0] + agg_ref[1]
    z = jnp.maximum(jnp.dot(z, wa_ref[...], preferred_element_type=jnp.float32), 0.0)
    z = jnp.maximum(jnp.dot(z, wb_ref[...], preferred_element_type=jnp.float32), 0.0)
    z = jnp.maximum(jnp.dot(z, wc_ref[...], preferred_element_type=jnp.float32), 0.0)

    @pl.when(i == 0)
    def _():
        acc_ref[...] = jnp.zeros_like(acc_ref)

    acc_ref[...] += jnp.sum(z, axis=0, keepdims=True)

    @pl.when(i == pl.num_programs(0) - 1)
    def _():
        head = jnp.dot(acc_ref[...] / N_NODES, l_ref[...],
                       preferred_element_type=jnp.float32)
        total = head + jnp.sum(prev_ref[...], axis=0, keepdims=True)
        out_ref[...] = jax.nn.sigmoid(total)


def _tc_mlp(h, agg, wa, wb, wc, l):
    grid = N_NODES // ROW_BLK
    return pl.pallas_call(
        _mlp_body,
        grid=(grid,),
        in_specs=[
            pl.BlockSpec((ROW_BLK, DIM), lambda i: (i, 0)),
            pl.BlockSpec((NC, ROW_BLK, DIM), lambda i: (0, i, 0)),
            pl.BlockSpec((DIM, DIM), lambda i: (0, 0)),
            pl.BlockSpec((DIM, DIM), lambda i: (0, 0)),
            pl.BlockSpec((DIM, DIM), lambda i: (0, 0)),
            pl.BlockSpec((DIM, 1), lambda i: (0, 0)),
        ],
        out_specs=[
            pl.BlockSpec((ROW_BLK, DIM), lambda i: (i, 0)),
            pl.BlockSpec((1, 1), lambda i: (0, 0)),
        ],
        out_shape=[
            jax.ShapeDtypeStruct((N_NODES, DIM), jnp.float32),
            jax.ShapeDtypeStruct((1, 1), jnp.float32),
        ],
        scratch_shapes=[pltpu.VMEM((1, DIM), jnp.float32)],
    )(h, agg, wa, wb, wc, l)


def _tc_final(h, agg, wa, wb, wc, l, prev):
    grid = N_NODES // ROW_BLK
    return pl.pallas_call(
        _final_body,
        grid=(grid,),
        in_specs=[
            pl.BlockSpec((ROW_BLK, DIM), lambda i: (i, 0)),
            pl.BlockSpec((NC, ROW_BLK, DIM), lambda i: (0, i, 0)),
            pl.BlockSpec((DIM, DIM), lambda i: (0, 0)),
            pl.BlockSpec((DIM, DIM), lambda i: (0, 0)),
            pl.BlockSpec((DIM, DIM), lambda i: (0, 0)),
            pl.BlockSpec((DIM, 1), lambda i: (0, 0)),
            pl.BlockSpec((4, 1), lambda i: (0, 0)),
        ],
        out_specs=pl.BlockSpec((1, 1), lambda i: (0, 0)),
        out_shape=jax.ShapeDtypeStruct((1, 1), jnp.float32),
        scratch_shapes=[pltpu.VMEM((1, DIM), jnp.float32)],
    )(h, agg, wa, wb, wc, l, prev)


def kernel(x, edge_index, Wa, Wb, Wc, L):
    src = edge_index[0]
    dst = edge_index[1]
    zeros = jnp.zeros((N_NODES, DIM), jnp.float32)

    h = x
    heads = []
    for i in range(4):
        agg = _sc_segsum(h, src, dst, zeros)
        h, head = _tc_mlp(h, agg, Wa[i], Wb[i], Wc[i], L[i])
        heads.append(head)

    agg = _sc_segsum(h, src, dst, zeros)
    prev = jnp.concatenate(heads, axis=0)
    out = _tc_final(h, agg, Wa[4], Wb[4], Wc[4], L[4], prev)
    return out.reshape((1,))
